# baseline (speedup 1.0000x reference)
import dataclasses
import functools

import numpy as np
import jax
import jax.numpy as jnp
from jax import lax
from jax.experimental import pallas as pl
from jax.experimental.pallas import tpu as pltpu
from jax.experimental.pallas import tpu_sc as plsc

_BINS = 10
_EPS = 1e-8
_B = 16384
_C = 1000
_NSUB = 16
_PW = _B // _NSUB
_NV = _PW // 16
_NG = _PW // 128


def _edges_f32():
    e = np.arange(_BINS + 1, dtype=np.float32) / np.float32(_BINS)
    e[-1] = np.float32(e[-1] + np.float32(1e-6))
    return [float(v) for v in e]


def _tc_dense_body(pred_ref, s_ref, ls_ref):
    x = pred_ref[...]
    s = jnp.sum(jnp.exp(x), axis=1)
    s_ref[0, 0, :] = s
    ls_ref[0, 0, :] = jnp.log(s + _EPS)


def _tc_dense(pred, row_block=512):
    bsz, csz = pred.shape
    nb = bsz // row_block
    s3, ls3 = pl.pallas_call(
        _tc_dense_body,
        grid=(nb,),
        in_specs=[pl.BlockSpec((row_block, csz), lambda i: (i, 0))],
        out_specs=[
            pl.BlockSpec((1, 1, row_block), lambda i: (i, 0, 0)),
            pl.BlockSpec((1, 1, row_block), lambda i: (i, 0, 0)),
        ],
        out_shape=[
            jax.ShapeDtypeStruct((nb, 1, row_block), jnp.float32),
            jax.ShapeDtypeStruct((nb, 1, row_block), jnp.float32),
        ],
    )(pred)
    return s3.reshape(bsz), ls3.reshape(bsz)


def _sc_body(pred_flat, tgt_hbm, s_hbm, ls_hbm, out_hbm,
             tgt_v, idx_v, m_v, s_v, ls_v, acc_v, gat_v, out_v,
             shared, sem):
    cid = lax.axis_index("c")
    sid = lax.axis_index("s")

    @pl.when(cid == 0)
    def _work():
        base = sid * _PW
        pltpu.sync_copy(tgt_hbm.at[pl.ds(base, _PW)], tgt_v)
        pltpu.sync_copy(s_hbm.at[pl.ds(base, _PW)], s_v)
        pltpu.sync_copy(ls_hbm.at[pl.ds(base, _PW)], ls_v)

        lane = lax.iota(jnp.int32, 16)

        @pl.loop(0, _NV)
        def _mkidx(k):
            t = tgt_v[pl.ds(k * 16, 16)]
            row = base + k * 16 + lane
            idx_v[pl.ds(k * 16, 16)] = row * _C + t

        copies = []
        for j in range(_NG):
            copies.append(pltpu.async_copy(
                pred_flat.at[idx_v.at[pl.ds(j * 128, 128)]],
                m_v.at[pl.ds(j * 128, 128)], sem))
        for cp in copies:
            cp.wait()

        zero = jnp.zeros((16,), jnp.float32)
        one = jnp.full((16,), 1.0, jnp.float32)
        for r in range(2 * _BINS):
            acc_v[r] = zero

        edges = _edges_f32()

        @pl.loop(0, _NV)
        def _binloop(k):
            sl = pl.ds(k * 16, 16)
            m = m_v[sl]
            s = s_v[sl]
            ls = ls_v[sl]
            g = 1.0 - jnp.exp(m) / s
            loss = ls - m
            nge = jnp.zeros((16,), jnp.int32)
            for ev in edges:
                nge = nge + jnp.where(g >= ev, 1, 0).astype(jnp.int32)
            bin_idx = jnp.minimum(jnp.maximum(nge - 1, 0), _BINS - 1)
            for b in range(_BINS):
                mb = bin_idx == b
                acc_v[b] = acc_v[b] + jnp.where(mb, 1.0, 0.0)
                acc_v[_BINS + b] = acc_v[_BINS + b] + jnp.where(mb, loss, 0.0)

        pltpu.sync_copy(acc_v, shared.at[pl.ds(sid * (2 * _BINS), 2 * _BINS)])

    plsc.subcore_barrier()

    @pl.when((cid == 0) & (sid == 0))
    def _final():
        pltpu.sync_copy(shared, gat_v)
        laneid = lax.iota(jnp.int32, 16)
        cnt16 = jnp.zeros((16,), jnp.float32)
        l16 = jnp.zeros((16,), jnp.float32)
        for b in range(_BINS):
            cv = gat_v[b]
            lv = gat_v[_BINS + b]
            for i in range(1, _NSUB):
                cv = cv + gat_v[i * (2 * _BINS) + b]
                lv = lv + gat_v[i * (2 * _BINS) + _BINS + b]
            cnt16 = jnp.where(laneid == b,
                              jnp.full((16,), jnp.sum(cv), jnp.float32), cnt16)
            l16 = jnp.where(laneid == b,
                            jnp.full((16,), jnp.sum(lv), jnp.float32), l16)
        mask = cnt16 > 0.0
        n = jnp.sum(jnp.where(mask, 1.0, 0.0))
        termv = jnp.where(mask, l16 / jnp.maximum(cnt16, 1.0), 0.0)
        tot = jnp.sum(termv)
        res_v = jnp.full((16,), tot, jnp.float32) / jnp.maximum(
            jnp.full((16,), n, jnp.float32), 1.0)
        out_v[...] = res_v
        pltpu.sync_copy(out_v, out_hbm)


def _sc_stage(pred_flat, target, s, ls):
    mesh = plsc.VectorSubcoreMesh(core_axis_name="c", subcore_axis_name="s")
    cp = pltpu.CompilerParams()
    if "needs_layout_passes" in pltpu.CompilerParams.__dataclass_fields__:
        cp = dataclasses.replace(cp, needs_layout_passes=False)
    k = pl.kernel(
        _sc_body,
        out_type=jax.ShapeDtypeStruct((16,), jnp.float32),
        mesh=mesh,
        scratch_types=[
            pltpu.VMEM((_PW,), jnp.int32),
            pltpu.VMEM((_PW,), jnp.int32),
            pltpu.VMEM((_PW,), jnp.float32),
            pltpu.VMEM((_PW,), jnp.float32),
            pltpu.VMEM((_PW,), jnp.float32),
            pltpu.VMEM((2 * _BINS, 16), jnp.float32),
            pltpu.VMEM((_NSUB * 2 * _BINS, 16), jnp.float32),
            pltpu.VMEM((16,), jnp.float32),
            pltpu.VMEM_SHARED((_NSUB * 2 * _BINS, 16), jnp.float32),
            pltpu.SemaphoreType.DMA,
        ],
        compiler_params=cp,
    )
    return k(pred_flat, target, s, ls)


def kernel(pred, target):
    s, ls = _tc_dense(pred)
    out16 = _sc_stage(pred.reshape(-1), target, s, ls)
    return out16[0]


# speedup vs baseline: 1.6133x; 1.6133x over previous
import dataclasses
import functools

import numpy as np
import jax
import jax.numpy as jnp
from jax import lax
from jax.experimental import pallas as pl
from jax.experimental.pallas import tpu as pltpu
from jax.experimental.pallas import tpu_sc as plsc

_BINS = 10
_EPS = 1e-8
_B = 16384
_C = 1000
_NSUB = 16
_PW = _B // _NSUB
_NV = _PW // 16


def _edges_f32():
    e = np.arange(_BINS + 1, dtype=np.float32) / np.float32(_BINS)
    e[-1] = np.float32(e[-1] + np.float32(1e-6))
    return [float(v) for v in e]


def _tc_dense_body(tgt_ref, pred_ref, g_ref, loss_ref):
    x = pred_ref[...]
    r, c = x.shape
    t = tgt_ref[0, 0, :]
    col = lax.broadcasted_iota(jnp.int32, (r, c), 1)
    e = jnp.exp(x)
    em = jnp.where(col == t[:, None], e, 0.0)
    s = jnp.sum(e, axis=1)
    em_s = jnp.sum(em, axis=1)
    m = jnp.log(em_s)
    ls = jnp.log(s + _EPS)
    g_ref[0, 0, :] = 1.0 - em_s / s
    loss_ref[0, 0, :] = ls - m


def _tc_dense(pred, target, row_block=512):
    bsz, csz = pred.shape
    nb = bsz // row_block
    tgt3 = target.reshape(nb, 1, row_block)
    g3, loss3 = pl.pallas_call(
        _tc_dense_body,
        grid=(nb,),
        in_specs=[
            pl.BlockSpec((1, 1, row_block), lambda i: (i, 0, 0)),
            pl.BlockSpec((row_block, csz), lambda i: (i, 0)),
        ],
        out_specs=[
            pl.BlockSpec((1, 1, row_block), lambda i: (i, 0, 0)),
            pl.BlockSpec((1, 1, row_block), lambda i: (i, 0, 0)),
        ],
        out_shape=[
            jax.ShapeDtypeStruct((nb, 1, row_block), jnp.float32),
            jax.ShapeDtypeStruct((nb, 1, row_block), jnp.float32),
        ],
    )(tgt3, pred)
    return g3.reshape(bsz), loss3.reshape(bsz)


_ACCR = 24


def _sc_body(g_hbm, loss_hbm, out_hbm, parts_hbm,
             g_v, l_v, acc_v, gat_v, out_v, sem):
    cid = lax.axis_index("c")
    sid = lax.axis_index("s")

    @pl.when(cid == 0)
    def _work():
        base = sid * _PW
        pltpu.sync_copy(g_hbm.at[pl.ds(base, _PW)], g_v)
        pltpu.sync_copy(loss_hbm.at[pl.ds(base, _PW)], l_v)

        zero = jnp.zeros((16,), jnp.float32)
        ones = jnp.full((16,), 1.0, jnp.float32)
        for r in range(_ACCR):
            acc_v[r] = zero

        edges = _edges_f32()

        @pl.loop(0, _NV)
        def _binloop(k):
            sl = pl.ds(k * 16, 16)
            g = g_v[sl]
            loss = l_v[sl]
            nge = jnp.zeros((16,), jnp.int32)
            for ev in edges:
                nge = nge + jnp.where(g >= ev, 1, 0).astype(jnp.int32)
            bin_idx = jnp.minimum(jnp.maximum(nge - 1, 0), _BINS - 1)
            plsc.addupdate_scatter(acc_v.at[0], [bin_idx], ones)
            plsc.addupdate_scatter(acc_v.at[1], [bin_idx], loss)

        pltpu.sync_copy(acc_v, parts_hbm.at[pl.ds(sid * _ACCR, _ACCR)])

    plsc.subcore_barrier()

    @pl.when((cid == 0) & (sid == 0))
    def _final():
        pltpu.sync_copy(parts_hbm, gat_v)
        cnt16 = gat_v[0]
        l16 = gat_v[1]
        for i in range(1, _NSUB):
            cnt16 = cnt16 + gat_v[i * _ACCR]
            l16 = l16 + gat_v[i * _ACCR + 1]
        mask = cnt16 > 0.0
        n = jnp.sum(jnp.where(mask, 1.0, 0.0))
        termv = jnp.where(mask, l16 / jnp.maximum(cnt16, 1.0), 0.0)
        tot = jnp.sum(termv)
        res_v = jnp.full((16,), tot, jnp.float32) / jnp.maximum(
            jnp.full((16,), n, jnp.float32), 1.0)
        out_v[...] = res_v
        pltpu.sync_copy(out_v, out_hbm)


def _sc_stage(g, loss):
    mesh = plsc.VectorSubcoreMesh(core_axis_name="c", subcore_axis_name="s")
    cp = pltpu.CompilerParams()
    if "needs_layout_passes" in pltpu.CompilerParams.__dataclass_fields__:
        cp = dataclasses.replace(cp, needs_layout_passes=False)
    k = pl.kernel(
        _sc_body,
        out_type=[
            jax.ShapeDtypeStruct((16,), jnp.float32),
            jax.ShapeDtypeStruct((_NSUB * _ACCR, 16), jnp.float32),
        ],
        mesh=mesh,
        scratch_types=[
            pltpu.VMEM((_PW,), jnp.float32),
            pltpu.VMEM((_PW,), jnp.float32),
            pltpu.VMEM((_ACCR, 16), jnp.float32),
            pltpu.VMEM((_NSUB * _ACCR, 16), jnp.float32),
            pltpu.VMEM((16,), jnp.float32),
            pltpu.SemaphoreType.DMA,
        ],
        compiler_params=cp,
    )
    out16, _parts = k(g, loss)
    return out16


def kernel(pred, target):
    g, loss = _tc_dense(pred, target)
    out16 = _sc_stage(g, loss)
    return out16[0]


# speedup vs baseline: 1.7044x; 1.0565x over previous
import dataclasses
import functools

import numpy as np
import jax
import jax.numpy as jnp
from jax import lax
from jax.experimental import pallas as pl
from jax.experimental.pallas import tpu as pltpu
from jax.experimental.pallas import tpu_sc as plsc

_BINS = 10
_EPS = 1e-8
_B = 16384
_C = 1000
_NSUB = 16
_PW = _B // _NSUB
_NV = _PW // 16


def _edges_f32():
    e = np.arange(_BINS + 1, dtype=np.float32) / np.float32(_BINS)
    e[-1] = np.float32(e[-1] + np.float32(1e-6))
    return [float(v) for v in e]


def _row_sum_via_transpose(a):
    r, c = a.shape
    nfull = c // 128
    acc = a[:, :128]
    for i in range(1, nfull):
        acc = acc + a[:, i * 128:(i + 1) * 128]
    tail = c - nfull * 128
    if tail:
        acc = acc + jnp.pad(a[:, nfull * 128:], ((0, 0), (0, 128 - tail)))
    return jnp.sum(acc.T, axis=0)


def _tc_dense_body(tgt_ref, pred_ref, g_ref, loss_ref):
    x = pred_ref[...]
    r, c = x.shape
    t = tgt_ref[0, 0, :]
    col = lax.broadcasted_iota(jnp.int32, (r, c), 1)
    e = jnp.exp(x)
    em = jnp.where(col == t[:, None], e, 0.0)
    s = _row_sum_via_transpose(e)
    em_s = _row_sum_via_transpose(em)
    m = jnp.log(em_s)
    ls = jnp.log(s + _EPS)
    g_ref[0, 0, :] = 1.0 - em_s / s
    loss_ref[0, 0, :] = ls - m


def _tc_dense(pred, target, row_block=512):
    bsz, csz = pred.shape
    nb = bsz // row_block
    tgt3 = target.reshape(nb, 1, row_block)
    g3, loss3 = pl.pallas_call(
        _tc_dense_body,
        grid=(nb,),
        in_specs=[
            pl.BlockSpec((1, 1, row_block), lambda i: (i, 0, 0)),
            pl.BlockSpec((row_block, csz), lambda i: (i, 0)),
        ],
        out_specs=[
            pl.BlockSpec((1, 1, row_block), lambda i: (i, 0, 0)),
            pl.BlockSpec((1, 1, row_block), lambda i: (i, 0, 0)),
        ],
        out_shape=[
            jax.ShapeDtypeStruct((nb, 1, row_block), jnp.float32),
            jax.ShapeDtypeStruct((nb, 1, row_block), jnp.float32),
        ],
    )(tgt3, pred)
    return g3.reshape(bsz), loss3.reshape(bsz)


_ACCR = 24


def _sc_body(g_hbm, loss_hbm, out_hbm, parts_hbm,
             g_v, l_v, acc_v, gat_v, out_v, sem):
    cid = lax.axis_index("c")
    sid = lax.axis_index("s")

    @pl.when(cid == 0)
    def _work():
        base = sid * _PW
        pltpu.sync_copy(g_hbm.at[pl.ds(base, _PW)], g_v)
        pltpu.sync_copy(loss_hbm.at[pl.ds(base, _PW)], l_v)

        zero = jnp.zeros((16,), jnp.float32)
        ones = jnp.full((16,), 1.0, jnp.float32)
        for r in range(_ACCR):
            acc_v[r] = zero

        edges = _edges_f32()

        @pl.loop(0, _NV)
        def _binloop(k):
            sl = pl.ds(k * 16, 16)
            g = g_v[sl]
            loss = l_v[sl]
            nge = jnp.zeros((16,), jnp.int32)
            for ev in edges:
                nge = nge + jnp.where(g >= ev, 1, 0).astype(jnp.int32)
            bin_idx = jnp.minimum(jnp.maximum(nge - 1, 0), _BINS - 1)
            plsc.addupdate_scatter(acc_v.at[0], [bin_idx], ones)
            plsc.addupdate_scatter(acc_v.at[1], [bin_idx], loss)

        pltpu.sync_copy(acc_v, parts_hbm.at[pl.ds(sid * _ACCR, _ACCR)])

    plsc.subcore_barrier()

    @pl.when((cid == 0) & (sid == 0))
    def _final():
        pltpu.sync_copy(parts_hbm, gat_v)
        cnt16 = gat_v[0]
        l16 = gat_v[1]
        for i in range(1, _NSUB):
            cnt16 = cnt16 + gat_v[i * _ACCR]
            l16 = l16 + gat_v[i * _ACCR + 1]
        mask = cnt16 > 0.0
        n = jnp.sum(jnp.where(mask, 1.0, 0.0))
        termv = jnp.where(mask, l16 / jnp.maximum(cnt16, 1.0), 0.0)
        tot = jnp.sum(termv)
        res_v = jnp.full((16,), tot, jnp.float32) / jnp.maximum(
            jnp.full((16,), n, jnp.float32), 1.0)
        out_v[...] = res_v
        pltpu.sync_copy(out_v, out_hbm)


def _sc_stage(g, loss):
    mesh = plsc.VectorSubcoreMesh(core_axis_name="c", subcore_axis_name="s")
    cp = pltpu.CompilerParams()
    if "needs_layout_passes" in pltpu.CompilerParams.__dataclass_fields__:
        cp = dataclasses.replace(cp, needs_layout_passes=False)
    k = pl.kernel(
        _sc_body,
        out_type=[
            jax.ShapeDtypeStruct((16,), jnp.float32),
            jax.ShapeDtypeStruct((_NSUB * _ACCR, 16), jnp.float32),
        ],
        mesh=mesh,
        scratch_types=[
            pltpu.VMEM((_PW,), jnp.float32),
            pltpu.VMEM((_PW,), jnp.float32),
            pltpu.VMEM((_ACCR, 16), jnp.float32),
            pltpu.VMEM((_NSUB * _ACCR, 16), jnp.float32),
            pltpu.VMEM((16,), jnp.float32),
            pltpu.SemaphoreType.DMA,
        ],
        compiler_params=cp,
    )
    out16, _parts = k(g, loss)
    return out16


def kernel(pred, target):
    g, loss = _tc_dense(pred, target)
    out16 = _sc_stage(g, loss)
    return out16[0]
